# baseline (speedup 1.0000x reference)
import functools

import jax
import jax.numpy as jnp
from jax import lax
from jax.experimental import pallas as pl
from jax.experimental.pallas import tpu as pltpu
from jax.experimental.pallas import tpu_sc as plsc

N_ATOMS_K = 262144
N_SPECIES_K = 118
DIM_K = 256
KEY_PAD = 128
LANES = 16
NUM_WORKERS = 32
B_PER_W = N_ATOMS_K // NUM_WORKERS
CHUNK = 128
N_CHUNKS = B_PER_W // CHUNK


def _sc_lookup_kernel(zs_hbm, zk_hbm, values_hbm, out_hbm, rep_hbm,
                      zs_v, zk_v, inv_v, idx_v, table_v, rows_v):
    wid = lax.axis_index("s") * 2 + lax.axis_index("c")
    base = wid * B_PER_W

    pltpu.sync_copy(zs_hbm.at[pl.ds(base, B_PER_W)], zs_v)
    pltpu.sync_copy(zk_hbm, zk_v)

    lanes = lax.iota(jnp.int32, LANES)
    for j in range(KEY_PAD // LANES):
        keys = zk_v[pl.ds(j * LANES, LANES)]
        plsc.store_scatter(inv_v, [keys], lanes + j * LANES)

    def map_body(i, carry):
        z = zs_v[pl.ds(i * LANES, LANES)]
        idx_v[pl.ds(i * LANES, LANES)] = (plsc.load_gather(inv_v, [z])
                                          + wid * KEY_PAD)
        return carry

    lax.fori_loop(0, B_PER_W // LANES, map_body, 0)

    pltpu.sync_copy(values_hbm, table_v)
    pltpu.sync_copy(table_v, rep_hbm.at[pl.ds(wid * KEY_PAD, KEY_PAD)])

    def idx_at(g):
        return idx_v.at[pl.ds(g * CHUNK, CHUNK)]

    def chunk_body(g, carry):
        pltpu.sync_copy(rep_hbm.at[idx_at(g)], rows_v)
        pltpu.sync_copy(rows_v, out_hbm.at[pl.ds(base + g * CHUNK, CHUNK)])
        return carry

    lax.fori_loop(0, N_CHUNKS, chunk_body, 0)


@jax.jit
def kernel(Zs, Z_keys, values):
    n_keys = Z_keys.shape[0]
    zk_pad = jnp.concatenate(
        [Z_keys.astype(jnp.int32),
         jnp.arange(n_keys, KEY_PAD, dtype=jnp.int32)])
    values_pad = jnp.pad(values, ((0, KEY_PAD - n_keys), (0, 0)))
    mesh = plsc.VectorSubcoreMesh(core_axis_name="c", subcore_axis_name="s")
    run = pl.kernel(
        _sc_lookup_kernel,
        mesh=mesh,
        compiler_params=pltpu.CompilerParams(needs_layout_passes=False),
        out_type=(
            jax.ShapeDtypeStruct((N_ATOMS_K, DIM_K), jnp.float32),
            jax.ShapeDtypeStruct((NUM_WORKERS * KEY_PAD, DIM_K),
                                 jnp.float32),
        ),
        scratch_types=[
            pltpu.VMEM((B_PER_W,), jnp.int32),
            pltpu.VMEM((KEY_PAD,), jnp.int32),
            pltpu.VMEM((KEY_PAD,), jnp.int32),
            pltpu.VMEM((B_PER_W,), jnp.int32),
            pltpu.VMEM((KEY_PAD, DIM_K), jnp.float32),
            pltpu.VMEM((CHUNK, DIM_K), jnp.float32),
        ],
    )
    return run(Zs, zk_pad, values_pad)[0]


# speedup vs baseline: 1.0241x; 1.0241x over previous
import functools

import jax
import jax.numpy as jnp
from jax import lax
from jax.experimental import pallas as pl
from jax.experimental.pallas import tpu as pltpu
from jax.experimental.pallas import tpu_sc as plsc

N_ATOMS_K = 262144
N_SPECIES_K = 118
DIM_K = 256
KEY_PAD = 128
LANES = 16
NUM_WORKERS = 32
B_PER_W = N_ATOMS_K // NUM_WORKERS
CHUNK = 128
N_CHUNKS = B_PER_W // CHUNK


def _sc_lookup_kernel(zs_hbm, zk_hbm, rep_hbm, out_hbm,
                      zs_v, zk_v, inv_v, idx_v, rows_v):
    wid = lax.axis_index("s") * 2 + lax.axis_index("c")
    base = wid * B_PER_W

    pltpu.sync_copy(zs_hbm.at[pl.ds(base, B_PER_W)], zs_v)
    pltpu.sync_copy(zk_hbm, zk_v)

    lanes = lax.iota(jnp.int32, LANES)
    for j in range(KEY_PAD // LANES):
        keys = zk_v[pl.ds(j * LANES, LANES)]
        plsc.store_scatter(inv_v, [keys], lanes + j * LANES)

    def map_body(i, carry):
        z = zs_v[pl.ds(i * LANES, LANES)]
        idx_v[pl.ds(i * LANES, LANES)] = (plsc.load_gather(inv_v, [z])
                                          + wid * KEY_PAD)
        return carry

    lax.fori_loop(0, B_PER_W // LANES, map_body, 0)


    def idx_at(g):
        return idx_v.at[pl.ds(g * CHUNK, CHUNK)]

    def chunk_body(g, carry):
        pltpu.sync_copy(rep_hbm.at[idx_at(g)], rows_v)
        pltpu.sync_copy(rows_v, out_hbm.at[pl.ds(base + g * CHUNK, CHUNK)])
        return carry

    lax.fori_loop(0, N_CHUNKS, chunk_body, 0)


@jax.jit
def kernel(Zs, Z_keys, values):
    n_keys = Z_keys.shape[0]
    zk_pad = jnp.concatenate(
        [Z_keys.astype(jnp.int32),
         jnp.arange(n_keys, KEY_PAD, dtype=jnp.int32)])
    values_pad = jnp.pad(values, ((0, KEY_PAD - n_keys), (0, 0)))
    mesh = plsc.VectorSubcoreMesh(core_axis_name="c", subcore_axis_name="s")
    run = pl.kernel(
        _sc_lookup_kernel,
        mesh=mesh,
        compiler_params=pltpu.CompilerParams(needs_layout_passes=False),
        out_type=jax.ShapeDtypeStruct((N_ATOMS_K, DIM_K), jnp.float32),
        scratch_types=[
            pltpu.VMEM((B_PER_W,), jnp.int32),
            pltpu.VMEM((KEY_PAD,), jnp.int32),
            pltpu.VMEM((KEY_PAD,), jnp.int32),
            pltpu.VMEM((B_PER_W,), jnp.int32),
            pltpu.VMEM((CHUNK, DIM_K), jnp.float32),
        ],
    )
    values_rep = jnp.tile(values_pad, (NUM_WORKERS, 1))
    return run(Zs, zk_pad, values_rep)


# speedup vs baseline: 1.2177x; 1.1890x over previous
import functools

import jax
import jax.numpy as jnp
from jax import lax
from jax.experimental import pallas as pl
from jax.experimental.pallas import tpu as pltpu
from jax.experimental.pallas import tpu_sc as plsc

N_ATOMS_K = 262144
N_SPECIES_K = 118
DIM_K = 256
KEY_PAD = 128
LANES = 16
NUM_WORKERS = 32
B_PER_W = N_ATOMS_K // NUM_WORKERS
CHUNK = 128
N_CHUNKS = B_PER_W // CHUNK


def _sc_lookup_kernel(zs_hbm, zk_hbm, rep_hbm, out_hbm,
                      zs_v, zk_v, inv_v, idx_v, rows_v, rows1_v,
                      gsem0, gsem1):
    wid = lax.axis_index("s") * 2 + lax.axis_index("c")
    base = wid * B_PER_W

    pltpu.sync_copy(zs_hbm.at[pl.ds(base, B_PER_W)], zs_v)
    pltpu.sync_copy(zk_hbm, zk_v)

    lanes = lax.iota(jnp.int32, LANES)
    for j in range(KEY_PAD // LANES):
        keys = zk_v[pl.ds(j * LANES, LANES)]
        plsc.store_scatter(inv_v, [keys], lanes + j * LANES)

    def map_body(i, carry):
        z = zs_v[pl.ds(i * LANES, LANES)]
        idx_v[pl.ds(i * LANES, LANES)] = (plsc.load_gather(inv_v, [z])
                                          + wid * KEY_PAD)
        return carry

    lax.fori_loop(0, B_PER_W // LANES, map_body, 0)


    def idx_at(g):
        return idx_v.at[pl.ds(g * CHUNK, CHUNK)]

    bufs = (rows_v, rows1_v)
    sems = (gsem0, gsem1)
    for b in range(2):
        pltpu.async_copy(rep_hbm.at[idx_at(b)], bufs[b], sems[b])

    def ring_body(h, carry):
        for b in range(2):
            g = 2 * h + b
            pltpu.make_async_copy(rep_hbm.at[idx_at(g)],
                                  bufs[b], sems[b]).wait()
            pltpu.sync_copy(bufs[b], out_hbm.at[pl.ds(base + g * CHUNK,
                                                      CHUNK)])

            @pl.when(g + 2 < N_CHUNKS)
            def _():
                pltpu.async_copy(rep_hbm.at[idx_at(g + 2)],
                                 bufs[b], sems[b])
        return carry

    lax.fori_loop(0, N_CHUNKS // 2, ring_body, 0)


@jax.jit
def kernel(Zs, Z_keys, values):
    n_keys = Z_keys.shape[0]
    zk_pad = jnp.concatenate(
        [Z_keys.astype(jnp.int32),
         jnp.arange(n_keys, KEY_PAD, dtype=jnp.int32)])
    values_pad = jnp.pad(values, ((0, KEY_PAD - n_keys), (0, 0)))
    mesh = plsc.VectorSubcoreMesh(core_axis_name="c", subcore_axis_name="s")
    run = pl.kernel(
        _sc_lookup_kernel,
        mesh=mesh,
        compiler_params=pltpu.CompilerParams(needs_layout_passes=False),
        out_type=jax.ShapeDtypeStruct((N_ATOMS_K, DIM_K), jnp.float32),
        scratch_types=[
            pltpu.VMEM((B_PER_W,), jnp.int32),
            pltpu.VMEM((KEY_PAD,), jnp.int32),
            pltpu.VMEM((KEY_PAD,), jnp.int32),
            pltpu.VMEM((B_PER_W,), jnp.int32),
            pltpu.VMEM((CHUNK, DIM_K), jnp.float32),
            pltpu.VMEM((CHUNK, DIM_K), jnp.float32),
            pltpu.SemaphoreType.DMA,
            pltpu.SemaphoreType.DMA,
        ],
    )
    values_rep = jnp.tile(values_pad, (NUM_WORKERS, 1))
    return run(Zs, zk_pad, values_rep)


# speedup vs baseline: 1.2187x; 1.0009x over previous
import functools

import jax
import jax.numpy as jnp
from jax import lax
from jax.experimental import pallas as pl
from jax.experimental.pallas import tpu as pltpu
from jax.experimental.pallas import tpu_sc as plsc

N_ATOMS_K = 262144
N_SPECIES_K = 118
DIM_K = 256
KEY_PAD = 128
LANES = 16
NUM_WORKERS = 32
B_PER_W = N_ATOMS_K // NUM_WORKERS
CHUNK = 128
N_CHUNKS = B_PER_W // CHUNK


def _sc_lookup_kernel(zs_hbm, zk_hbm, rep_hbm, out_hbm,
                      zs_v, zk_v, inv_v, idx_v, rows_v, rows1_v, rows2_v,
                      gsem0, gsem1, gsem2, wsem0, wsem1, wsem2):
    wid = lax.axis_index("s") * 2 + lax.axis_index("c")
    base = wid * B_PER_W

    pltpu.sync_copy(zs_hbm.at[pl.ds(base, B_PER_W)], zs_v)
    pltpu.sync_copy(zk_hbm, zk_v)

    lanes = lax.iota(jnp.int32, LANES)
    for j in range(KEY_PAD // LANES):
        keys = zk_v[pl.ds(j * LANES, LANES)]
        plsc.store_scatter(inv_v, [keys], lanes + j * LANES)

    def map_body(i, carry):
        z = zs_v[pl.ds(i * LANES, LANES)]
        idx_v[pl.ds(i * LANES, LANES)] = (plsc.load_gather(inv_v, [z])
                                          + wid * KEY_PAD)
        return carry

    lax.fori_loop(0, B_PER_W // LANES, map_body, 0)


    def idx_at(g):
        return idx_v.at[pl.ds(g * CHUNK, CHUNK)]

    bufs = (rows_v, rows1_v, rows2_v)
    gsems = (gsem0, gsem1, gsem2)
    wsems = (wsem0, wsem1, wsem2)

    def out_at(g):
        return out_hbm.at[pl.ds(base + g * CHUNK, CHUNK)]

    for b in range(2):
        pltpu.async_copy(rep_hbm.at[idx_at(b)], bufs[b], gsems[b])

    def ring_body(h, carry):
        for b in range(3):
            g = 3 * h + b
            pltpu.make_async_copy(rep_hbm.at[idx_at(g)],
                                  bufs[b], gsems[b]).wait()
            pltpu.async_copy(bufs[b], out_at(g), wsems[b])
            b2 = (b + 2) % 3

            @pl.when(g + 2 < N_CHUNKS)
            def _():
                @pl.when(g >= 1)
                def _():
                    pltpu.make_async_copy(bufs[b2], out_at(g - 1),
                                          wsems[b2]).wait()
                pltpu.async_copy(rep_hbm.at[idx_at(g + 2)],
                                 bufs[b2], gsems[b2])
        return carry

    lax.fori_loop(0, N_CHUNKS // 3, ring_body, 0)

    g_last = N_CHUNKS - 1
    b_last = g_last % 3
    pltpu.make_async_copy(rep_hbm.at[idx_at(g_last)],
                          bufs[b_last], gsems[b_last]).wait()
    pltpu.async_copy(bufs[b_last], out_at(g_last), wsems[b_last])
    for g in (N_CHUNKS - 3, N_CHUNKS - 2, N_CHUNKS - 1):
        pltpu.make_async_copy(bufs[g % 3], out_at(g), wsems[g % 3]).wait()


@jax.jit
def kernel(Zs, Z_keys, values):
    n_keys = Z_keys.shape[0]
    zk_pad = jnp.concatenate(
        [Z_keys.astype(jnp.int32),
         jnp.arange(n_keys, KEY_PAD, dtype=jnp.int32)])
    values_pad = jnp.pad(values, ((0, KEY_PAD - n_keys), (0, 0)))
    mesh = plsc.VectorSubcoreMesh(core_axis_name="c", subcore_axis_name="s")
    run = pl.kernel(
        _sc_lookup_kernel,
        mesh=mesh,
        compiler_params=pltpu.CompilerParams(needs_layout_passes=False),
        out_type=jax.ShapeDtypeStruct((N_ATOMS_K, DIM_K), jnp.float32),
        scratch_types=[
            pltpu.VMEM((B_PER_W,), jnp.int32),
            pltpu.VMEM((KEY_PAD,), jnp.int32),
            pltpu.VMEM((KEY_PAD,), jnp.int32),
            pltpu.VMEM((B_PER_W,), jnp.int32),
            pltpu.VMEM((CHUNK, DIM_K), jnp.float32),
            pltpu.VMEM((CHUNK, DIM_K), jnp.float32),
            pltpu.VMEM((CHUNK, DIM_K), jnp.float32),
            pltpu.SemaphoreType.DMA,
            pltpu.SemaphoreType.DMA,
            pltpu.SemaphoreType.DMA,
            pltpu.SemaphoreType.DMA,
            pltpu.SemaphoreType.DMA,
            pltpu.SemaphoreType.DMA,
        ],
    )
    values_rep = jnp.tile(values_pad, (NUM_WORKERS, 1))
    return run(Zs, zk_pad, values_rep)
